# baseline (speedup 1.0000x reference)
import functools

import jax
import jax.numpy as jnp
from jax import lax
from jax.experimental import pallas as pl
from jax.experimental.pallas import tpu as pltpu
from jax.experimental.pallas import tpu_sc as plsc

N = 10000
E = 320000
D = 128
T = 16
DH = D // 2
NP = 10240
NC = 2
NS = 16
NW = NC * NS
RW = 80
ROWS = E // RW
RPT = ROWS // NS
SCH = 5
CHUNKS = RPT // SCH
CE = SCH * RW
NPT = N // NS

_mesh = plsc.VectorSubcoreMesh(core_axis_name="c", subcore_axis_name="s")
_sc_params = pltpu.CompilerParams(use_tc_tiling_on_sc=False,
                                  needs_layout_passes=False)


@functools.partial(
    pl.kernel,
    out_type=jax.ShapeDtypeStruct((NC, 2, NP), jnp.float32),
    mesh=_mesh,
    scratch_types=[
        pltpu.VMEM_SHARED((NP,), jnp.float32),
        pltpu.VMEM_SHARED((NP,), jnp.float32),
        pltpu.VMEM((SCH, RW), jnp.int32),
        pltpu.VMEM((RW,), jnp.float32),
        pltpu.VMEM((NP // NS,), jnp.float32),
    ],
    compiler_params=_sc_params,
)
def _hist(es_hbm, ed_hbm, out_hbm, hs_sp, hd_sp, idxb, onesb, zb):
    c = lax.axis_index("c")
    s = lax.axis_index("s")
    wid = s * NC + c
    for i in range(RW // 16):
        onesb[pl.ds(16 * i, 16)] = jnp.ones((16,), jnp.float32)
    for i in range(NP // NS // 16):
        zb[pl.ds(16 * i, 16)] = jnp.zeros((16,), jnp.float32)
    zoff = s * (NP // NS)
    pltpu.sync_copy(zb, hs_sp.at[pl.ds(zoff, NP // NS)])
    pltpu.sync_copy(zb, hd_sp.at[pl.ds(zoff, NP // NS)])
    plsc.subcore_barrier()

    def chunk(k, carry):
        rb = wid * (ROWS // NW) + k * SCH
        pltpu.sync_copy(es_hbm.at[pl.ds(rb, SCH)], idxb)
        for i in range(SCH):
            pltpu.sync_copy(onesb, hs_sp.at[idxb.at[i]], add=True)
        pltpu.sync_copy(ed_hbm.at[pl.ds(rb, SCH)], idxb)
        for i in range(SCH):
            pltpu.sync_copy(onesb, hd_sp.at[idxb.at[i]], add=True)
        return carry

    lax.fori_loop(0, ROWS // NW // SCH, chunk, 0)
    plsc.subcore_barrier()
    pltpu.sync_copy(hs_sp.at[pl.ds(zoff, NP // NS)],
                    out_hbm.at[c, 0, pl.ds(zoff, NP // NS)])
    pltpu.sync_copy(hd_sp.at[pl.ds(zoff, NP // NS)],
                    out_hbm.at[c, 1, pl.ds(zoff, NP // NS)])


_RB = 2000


def _prep_body(x_ref, ws_ref, wd_ref, bs_ref, bd_ref, hist_ref,
               ht_out, inv_out):
    x = x_ref[...]
    hs = jnp.dot(x, ws_ref[...], preferred_element_type=jnp.float32) \
        + bs_ref[...]
    hd = jnp.dot(x, wd_ref[...], preferred_element_type=jnp.float32) \
        + bd_ref[...]
    ht_out[0, 0] = hs[:, :DH]
    ht_out[0, 1] = hd[:, :DH]
    ht_out[1, 0] = hs[:, DH:]
    ht_out[1, 1] = hd[:, DH:]

    @pl.when(pl.program_id(0) == 0)
    def _():
        deg = hist_ref[0] + hist_ref[1]
        inv_out[...] = lax.rsqrt(jnp.maximum(deg, 1.0))


_prep = pl.pallas_call(
    _prep_body,
    grid=(N // _RB,),
    in_specs=[
        pl.BlockSpec((_RB, D), lambda i: (i, 0)),
        pl.BlockSpec((D, D), lambda i: (0, 0)),
        pl.BlockSpec((D, D), lambda i: (0, 0)),
        pl.BlockSpec((1, D), lambda i: (0, 0)),
        pl.BlockSpec((1, D), lambda i: (0, 0)),
        pl.BlockSpec((NC, 2, NP), lambda i: (0, 0, 0)),
    ],
    out_specs=[
        pl.BlockSpec((NC, 2, _RB, DH), lambda i: (0, 0, i, 0)),
        pl.BlockSpec((2, NP), lambda i: (0, 0)),
    ],
    out_shape=[
        jax.ShapeDtypeStruct((NC, 2, N, DH), jnp.float32),
        jax.ShapeDtypeStruct((2, NP), jnp.float32),
    ],
)


SCH2 = 2
CE2 = SCH2 * RW
CH2 = RPT // SCH2
GPC = CE2 // 16


@functools.partial(
    pl.kernel,
    out_type=jax.ShapeDtypeStruct((NC, N, DH), jnp.float32),
    mesh=_mesh,
    scratch_types=(
        [pltpu.VMEM((SCH2, RW), jnp.int32) for _ in range(2)]
        + [pltpu.VMEM((SCH2, RW), jnp.int32) for _ in range(2)]
        + [pltpu.VMEM((SCH2, RW), jnp.int32) for _ in range(4)]
        + [pltpu.VMEM((SCH2, RW), jnp.int32) for _ in range(2)]
        + [pltpu.VMEM((CE2,), jnp.int32) for _ in range(2)]
        + [pltpu.VMEM((CE2,), jnp.float32) for _ in range(2)]
        + [pltpu.VMEM((CE2, DH), jnp.float32) for _ in range(2)]
        + [pltpu.VMEM((CE2, DH), jnp.float32) for _ in range(2)]
        + [pltpu.VMEM((CE2, DH), jnp.float32) for _ in range(2)]
        + [pltpu.VMEM((T * DH,), jnp.float32)]
        + [pltpu.VMEM_SHARED((N, DH), jnp.float32)]
        + [pltpu.SemaphoreType.DMA for _ in range(4)]
    ),
    compiler_params=_sc_params,
)
def _main(ht_hbm, es_hbm, ed_hbm, ec_hbm, inv_hbm, emb_hbm, out_hbm,
          sidx0, sidx1, sidxa0, sidxa1, didx0, didx1, didx2_, didx3,
          didxa0, didxa1, cvec0, cvec1, invv0, invv1,
          srcb0, srcb1, dstb0, dstb1, msgb0, msgb1, embv, acc,
          semg0, semg1, sems0, sems1):
    SIDX = [sidx0, sidx1]
    SIDXA = [sidxa0, sidxa1]
    DIDX = [didx0, didx1, didx2_, didx3]
    DIDXA = [didxa0, didxa1]
    CVEC = [cvec0, cvec1]
    INVV = [invv0, invv1]
    SRCB = [srcb0, srcb1]
    DSTB = [dstb0, dstb1]
    MSGB = [msgb0, msgb1]
    SEMG = [semg0, semg1]
    SEMS = [sems0, sems1]

    c = lax.axis_index("c")
    s = lax.axis_index("s")
    pltpu.sync_copy(emb_hbm.at[c], embv)

    def zr(r, carry):
        for j in range(DH // 16):
            msgb0[r, pl.ds(16 * j, 16)] = jnp.zeros((16,), jnp.float32)
        return carry

    lax.fori_loop(0, CE2, zr, 0)
    base_n = s * NPT
    for r0 in range(0, NPT - CE2 + 1, CE2):
        pltpu.sync_copy(msgb0.at[pl.ds(0, CE2)],
                        acc.at[pl.ds(base_n + r0, CE2)])
    rem = NPT % CE2
    if rem:
        pltpu.sync_copy(msgb0.at[pl.ds(0, rem)],
                        acc.at[pl.ds(base_n + NPT - rem, rem)])
    plsc.subcore_barrier()

    iota1 = lax.iota(jnp.int32, 16)
    soff = c * (2 * N)
    doff = c * (2 * N) + N

    def load_and_gather(k, p, d4):
        rb = s * RPT + k * SCH2
        eb = rb * RW
        pltpu.sync_copy(es_hbm.at[pl.ds(rb, SCH2)], SIDX[p])
        pltpu.sync_copy(ed_hbm.at[pl.ds(rb, SCH2)], DIDX[d4])
        pltpu.sync_copy(ec_hbm.at[pl.ds(eb, CE2)], CVEC[p])
        for i in range(SCH2):
            for q in range(RW // 16):
                SIDXA[p][i, pl.ds(16 * q, 16)] = \
                    SIDX[p][i, pl.ds(16 * q, 16)] + soff
                DIDXA[p][i, pl.ds(16 * q, 16)] = \
                    DIDX[d4][i, pl.ds(16 * q, 16)] + doff
        for i in range(SCH2):
            pltpu.async_copy(ht_hbm.at[SIDXA[p].at[i]],
                             SRCB[p].at[pl.ds(RW * i, RW)], SEMG[p])
            pltpu.async_copy(ht_hbm.at[DIDXA[p].at[i]],
                             DSTB[p].at[pl.ds(RW * i, RW)], SEMG[p])
            pltpu.async_copy(inv_hbm.at[SIDX[p].at[i]],
                             INVV[p].at[pl.ds(RW * i, RW)], SEMG[p])

    def wait_gathers(p):
        for i in range(SCH2):
            pltpu.make_async_copy(ht_hbm.at[SIDXA[p].at[i]],
                                  SRCB[p].at[pl.ds(RW * i, RW)],
                                  SEMG[p]).wait()
            pltpu.make_async_copy(ht_hbm.at[DIDXA[p].at[i]],
                                  DSTB[p].at[pl.ds(RW * i, RW)],
                                  SEMG[p]).wait()
            pltpu.make_async_copy(inv_hbm.at[SIDX[p].at[i]],
                                  INVV[p].at[pl.ds(RW * i, RW)],
                                  SEMG[p]).wait()

    def drain_scatters(p, d4):
        for i in range(SCH2):
            pltpu.make_async_copy(MSGB[p].at[pl.ds(RW * i, RW)],
                                  acc.at[DIDX[d4].at[i]], SEMS[p]).wait()

    iotas = [iota1 + 16 * j for j in range(DH // 16)]

    def bcast(vec, r):
        idx = jnp.full((16, 1), r, jnp.int32)
        return lax.gather(
            vec, idx,
            lax.GatherDimensionNumbers(offset_dims=(),
                                       collapsed_slice_dims=(0,),
                                       start_index_map=(0,)),
            (1,), mode=lax.GatherScatterMode.PROMISE_IN_BOUNDS)

    def compute(p):
        def group(g, gc):
            inv_v = INVV[p][pl.ds(16 * g, 16)]
            cls_v = CVEC[p][pl.ds(16 * g, 16)] * DH
            for r in range(16):
                inv_b = bcast(inv_v, r)
                cls_b = bcast(cls_v, r)
                b = 16 * g + r
                for j in range(DH // 16):
                    sv = SRCB[p][b, pl.ds(16 * j, 16)]
                    dv = DSTB[p][b, pl.ds(16 * j, 16)]
                    ev = plsc.load_gather(embv, [cls_b + iotas[j]])
                    m = jnp.maximum(sv + dv + ev, 0.0) * inv_b
                    MSGB[p][b, pl.ds(16 * j, 16)] = m
            return gc

        lax.fori_loop(0, GPC, group, 0)

    def fire_scatters(p, d4):
        for i in range(SCH2):
            pltpu.async_copy(MSGB[p].at[pl.ds(RW * i, RW)],
                             acc.at[DIDX[d4].at[i]], SEMS[p], add=True)

    load_and_gather(0, 0, 0)

    def quad(kk, carry):
        for p in range(4):
            k = kk * 4 + p
            p2 = p % 2
            wait_gathers(p2)
            if p >= 2:
                drain_scatters(p2, (p - 2) % 4)
            else:
                @pl.when(kk >= 1)
                def _():
                    drain_scatters(p2, (p + 2) % 4)
            load_and_gather(k + 1, 1 - p2, (p + 1) % 4)
            compute(p2)
            fire_scatters(p2, p)
        return carry

    lax.fori_loop(0, CH2 // 4, quad, 0)
    wait_gathers(0)
    drain_scatters(0, 2)
    compute(0)
    fire_scatters(0, 0)
    drain_scatters(1, 3)
    drain_scatters(0, 0)
    plsc.subcore_barrier()
    pltpu.sync_copy(acc.at[pl.ds(base_n, NPT)],
                    out_hbm.at[c, pl.ds(base_n, NPT)])


def _final_body(p_ref, invd_ref, o_ref):
    inv = invd_ref[...]
    o_ref[...] = jnp.concatenate([p_ref[0] * inv, p_ref[1] * inv], axis=1)


_final = pl.pallas_call(
    _final_body,
    grid=(N // _RB,),
    in_specs=[
        pl.BlockSpec((NC, _RB, DH), lambda i: (0, i, 0)),
        pl.BlockSpec((_RB, 1), lambda i: (i, 0)),
    ],
    out_specs=pl.BlockSpec((_RB, D), lambda i: (i, 0)),
    out_shape=jax.ShapeDtypeStruct((N, D), jnp.float32),
)


def kernel(x, edge_src, edge_dst, edge_classes, W_src, b_src, W_dst, b_dst,
           edge_emb):
    es2 = edge_src.reshape(ROWS, RW)
    ed2 = edge_dst.reshape(ROWS, RW)
    hist = _hist(es2, ed2)
    ht, invs = _prep(x, W_src, W_dst, b_src.reshape(1, D),
                     b_dst.reshape(1, D), hist)
    inv_ns = invs[0]
    inv_nd = invs[1, :N].reshape(N, 1)
    emb2 = edge_emb.reshape(T, NC, DH).transpose(1, 0, 2).reshape(NC, T * DH)
    parts = _main(ht.reshape(NC * 2 * N, DH), es2, ed2, edge_classes,
                  inv_ns, emb2)
    return _final(parts, inv_nd)


# speedup vs baseline: 1.2571x; 1.2571x over previous
import functools

import jax
import jax.numpy as jnp
from jax import lax
from jax.experimental import pallas as pl
from jax.experimental.pallas import tpu as pltpu
from jax.experimental.pallas import tpu_sc as plsc

N = 10000
E = 320000
D = 128
T = 16
DH = D // 2
NP = 10240
NC = 2
NS = 16
NW = NC * NS
RW = 80
ROWS = E // RW
RPT = ROWS // NS
SCH = 5
CHUNKS = RPT // SCH
CE = SCH * RW
NPT = N // NS

_mesh = plsc.VectorSubcoreMesh(core_axis_name="c", subcore_axis_name="s")
_sc_params = pltpu.CompilerParams(use_tc_tiling_on_sc=False,
                                  needs_layout_passes=False)


@functools.partial(
    pl.kernel,
    out_type=jax.ShapeDtypeStruct((NC, 2, NP), jnp.float32),
    mesh=_mesh,
    scratch_types=[
        pltpu.VMEM_SHARED((NP,), jnp.float32),
        pltpu.VMEM_SHARED((NP,), jnp.float32),
        pltpu.VMEM((SCH, RW), jnp.int32),
        pltpu.VMEM((RW,), jnp.float32),
        pltpu.VMEM((NP // NS,), jnp.float32),
    ],
    compiler_params=_sc_params,
)
def _hist(es_hbm, ed_hbm, out_hbm, hs_sp, hd_sp, idxb, onesb, zb):
    c = lax.axis_index("c")
    s = lax.axis_index("s")
    wid = s * NC + c
    for i in range(RW // 16):
        onesb[pl.ds(16 * i, 16)] = jnp.ones((16,), jnp.float32)
    for i in range(NP // NS // 16):
        zb[pl.ds(16 * i, 16)] = jnp.zeros((16,), jnp.float32)
    zoff = s * (NP // NS)
    pltpu.sync_copy(zb, hs_sp.at[pl.ds(zoff, NP // NS)])
    pltpu.sync_copy(zb, hd_sp.at[pl.ds(zoff, NP // NS)])
    plsc.subcore_barrier()

    def chunk(k, carry):
        rb = wid * (ROWS // NW) + k * SCH
        pltpu.sync_copy(es_hbm.at[pl.ds(rb, SCH)], idxb)
        for i in range(SCH):
            pltpu.sync_copy(onesb, hs_sp.at[idxb.at[i]], add=True)
        pltpu.sync_copy(ed_hbm.at[pl.ds(rb, SCH)], idxb)
        for i in range(SCH):
            pltpu.sync_copy(onesb, hd_sp.at[idxb.at[i]], add=True)
        return carry

    lax.fori_loop(0, ROWS // NW // SCH, chunk, 0)
    plsc.subcore_barrier()
    pltpu.sync_copy(hs_sp.at[pl.ds(zoff, NP // NS)],
                    out_hbm.at[c, 0, pl.ds(zoff, NP // NS)])
    pltpu.sync_copy(hd_sp.at[pl.ds(zoff, NP // NS)],
                    out_hbm.at[c, 1, pl.ds(zoff, NP // NS)])


_RB = 2000


def _prep_body(x_ref, ws_ref, wd_ref, bs_ref, bd_ref, hist_ref,
               ht_out, inv_out):
    x = x_ref[...]
    hs = jnp.dot(x, ws_ref[...], preferred_element_type=jnp.float32) \
        + bs_ref[...]
    hd = jnp.dot(x, wd_ref[...], preferred_element_type=jnp.float32) \
        + bd_ref[...]
    ht_out[0, 0] = hs[:, :DH]
    ht_out[0, 1] = hd[:, :DH]
    ht_out[1, 0] = hs[:, DH:]
    ht_out[1, 1] = hd[:, DH:]

    @pl.when(pl.program_id(0) == 0)
    def _():
        deg = hist_ref[0] + hist_ref[1]
        inv_out[...] = lax.rsqrt(jnp.maximum(deg, 1.0))


_prep = pl.pallas_call(
    _prep_body,
    grid=(N // _RB,),
    in_specs=[
        pl.BlockSpec((_RB, D), lambda i: (i, 0)),
        pl.BlockSpec((D, D), lambda i: (0, 0)),
        pl.BlockSpec((D, D), lambda i: (0, 0)),
        pl.BlockSpec((1, D), lambda i: (0, 0)),
        pl.BlockSpec((1, D), lambda i: (0, 0)),
        pl.BlockSpec((NC, 2, NP), lambda i: (0, 0, 0)),
    ],
    out_specs=[
        pl.BlockSpec((NC, 2, _RB, DH), lambda i: (0, 0, i, 0)),
        pl.BlockSpec((2, NP), lambda i: (0, 0)),
    ],
    out_shape=[
        jax.ShapeDtypeStruct((NC, 2, N, DH), jnp.float32),
        jax.ShapeDtypeStruct((2, NP), jnp.float32),
    ],
)


SCH2 = 2
CE2 = SCH2 * RW
CH2 = RPT // SCH2
GPC = CE2 // 16


@functools.partial(
    pl.kernel,
    out_type=jax.ShapeDtypeStruct((NC, N, DH), jnp.float32),
    mesh=_mesh,
    scratch_types=(
        [pltpu.VMEM((SCH2, RW), jnp.int32) for _ in range(2)]
        + [pltpu.VMEM((SCH2, RW), jnp.int32) for _ in range(2)]
        + [pltpu.VMEM((SCH2, RW), jnp.int32) for _ in range(4)]
        + [pltpu.VMEM((SCH2, RW), jnp.int32) for _ in range(2)]
        + [pltpu.VMEM((CE2,), jnp.int32) for _ in range(4)]
        + [pltpu.VMEM((CE2,), jnp.float32) for _ in range(2)]
        + [pltpu.VMEM((CE2, DH), jnp.float32) for _ in range(2)]
        + [pltpu.VMEM((CE2, DH), jnp.float32) for _ in range(2)]
        + [pltpu.VMEM((CE2, DH), jnp.float32) for _ in range(2)]
        + [pltpu.VMEM((T * DH,), jnp.float32)]
        + [pltpu.VMEM_SHARED((N, DH), jnp.float32)]
        + [pltpu.SemaphoreType.DMA for _ in range(6)]
    ),
    compiler_params=_sc_params,
)
def _main(ht_hbm, es_hbm, ed_hbm, ec_hbm, inv_hbm, emb_hbm, out_hbm,
          sidx0, sidx1, sidxa0, sidxa1, didx0, didx1, didx2_, didx3,
          didxa0, didxa1, cvec0, cvec1, cvec2, cvec3, invv0, invv1,
          srcb0, srcb1, dstb0, dstb1, msgb0, msgb1, embv, acc,
          semg0, semg1, sems0, sems1, semi0, semi1):
    SIDX = [sidx0, sidx1]
    SIDXA = [sidxa0, sidxa1]
    DIDX = [didx0, didx1, didx2_, didx3]
    DIDXA = [didxa0, didxa1]
    CVEC = [cvec0, cvec1, cvec2, cvec3]
    INVV = [invv0, invv1]
    SRCB = [srcb0, srcb1]
    DSTB = [dstb0, dstb1]
    MSGB = [msgb0, msgb1]
    SEMG = [semg0, semg1]
    SEMS = [sems0, sems1]
    SEMI = [semi0, semi1]

    c = lax.axis_index("c")
    s = lax.axis_index("s")
    pltpu.sync_copy(emb_hbm.at[c], embv)

    def zr(r, carry):
        for j in range(DH // 16):
            msgb0[r, pl.ds(16 * j, 16)] = jnp.zeros((16,), jnp.float32)
        return carry

    lax.fori_loop(0, CE2, zr, 0)
    base_n = s * NPT
    for r0 in range(0, NPT - CE2 + 1, CE2):
        pltpu.sync_copy(msgb0.at[pl.ds(0, CE2)],
                        acc.at[pl.ds(base_n + r0, CE2)])
    rem = NPT % CE2
    if rem:
        pltpu.sync_copy(msgb0.at[pl.ds(0, rem)],
                        acc.at[pl.ds(base_n + NPT - rem, rem)])
    plsc.subcore_barrier()

    iota1 = lax.iota(jnp.int32, 16)
    soff = c * (2 * N)
    doff = c * (2 * N) + N

    def fire_idx(k, pi, d4):
        rb = s * RPT + k * SCH2
        eb = rb * RW
        pltpu.async_copy(es_hbm.at[pl.ds(rb, SCH2)], SIDX[pi], SEMI[pi])
        pltpu.async_copy(ed_hbm.at[pl.ds(rb, SCH2)], DIDX[d4], SEMI[pi])
        pltpu.async_copy(ec_hbm.at[pl.ds(eb, CE2)], CVEC[d4], SEMI[pi])

    def wait_idx(pi, d4):
        pltpu.make_async_copy(es_hbm.at[pl.ds(0, SCH2)],
                              SIDX[pi], SEMI[pi]).wait()
        pltpu.make_async_copy(ed_hbm.at[pl.ds(0, SCH2)],
                              DIDX[d4], SEMI[pi]).wait()
        pltpu.make_async_copy(ec_hbm.at[pl.ds(0, CE2)],
                              CVEC[d4], SEMI[pi]).wait()

    def adjust_and_gather(p, d4):
        for i in range(SCH2):
            for q in range(RW // 16):
                SIDXA[p][i, pl.ds(16 * q, 16)] = \
                    SIDX[p][i, pl.ds(16 * q, 16)] + soff
                DIDXA[p][i, pl.ds(16 * q, 16)] = \
                    DIDX[d4][i, pl.ds(16 * q, 16)] + doff
        for i in range(SCH2):
            pltpu.async_copy(ht_hbm.at[SIDXA[p].at[i]],
                             SRCB[p].at[pl.ds(RW * i, RW)], SEMG[p])
            pltpu.async_copy(ht_hbm.at[DIDXA[p].at[i]],
                             DSTB[p].at[pl.ds(RW * i, RW)], SEMG[p])
            pltpu.async_copy(inv_hbm.at[SIDX[p].at[i]],
                             INVV[p].at[pl.ds(RW * i, RW)], SEMG[p])

    def wait_gathers(p):
        for i in range(SCH2):
            pltpu.make_async_copy(ht_hbm.at[SIDXA[p].at[i]],
                                  SRCB[p].at[pl.ds(RW * i, RW)],
                                  SEMG[p]).wait()
            pltpu.make_async_copy(ht_hbm.at[DIDXA[p].at[i]],
                                  DSTB[p].at[pl.ds(RW * i, RW)],
                                  SEMG[p]).wait()
            pltpu.make_async_copy(inv_hbm.at[SIDX[p].at[i]],
                                  INVV[p].at[pl.ds(RW * i, RW)],
                                  SEMG[p]).wait()

    def drain_scatters(p, d4):
        for i in range(SCH2):
            pltpu.make_async_copy(MSGB[p].at[pl.ds(RW * i, RW)],
                                  acc.at[DIDX[d4].at[i]], SEMS[p]).wait()

    iotas = [iota1 + 16 * j for j in range(DH // 16)]

    def bcast(vec, r):
        idx = jnp.full((16, 1), r, jnp.int32)
        return lax.gather(
            vec, idx,
            lax.GatherDimensionNumbers(offset_dims=(),
                                       collapsed_slice_dims=(0,),
                                       start_index_map=(0,)),
            (1,), mode=lax.GatherScatterMode.PROMISE_IN_BOUNDS)

    def compute(p, c4):
        def group(g, gc):
            inv_v = INVV[p][pl.ds(16 * g, 16)]
            cls_v = CVEC[c4][pl.ds(16 * g, 16)] * DH
            for r in range(16):
                inv_b = bcast(inv_v, r)
                cls_b = bcast(cls_v, r)
                b = 16 * g + r
                for j in range(DH // 16):
                    sv = SRCB[p][b, pl.ds(16 * j, 16)]
                    dv = DSTB[p][b, pl.ds(16 * j, 16)]
                    ev = plsc.load_gather(embv, [cls_b + iotas[j]])
                    m = jnp.maximum(sv + dv + ev, 0.0) * inv_b
                    MSGB[p][b, pl.ds(16 * j, 16)] = m
            return gc

        lax.fori_loop(0, GPC, group, 0)

    def fire_scatters(p, d4):
        for i in range(SCH2):
            pltpu.async_copy(MSGB[p].at[pl.ds(RW * i, RW)],
                             acc.at[DIDX[d4].at[i]], SEMS[p], add=True)

    fire_idx(0, 0, 0)
    wait_idx(0, 0)
    adjust_and_gather(0, 0)
    fire_idx(1, 1, 1)

    def quad(kk, carry):
        for p in range(4):
            k = kk * 4 + p
            p2 = p % 2
            wait_gathers(p2)
            if p >= 2:
                drain_scatters(p2, (p - 2) % 4)
            else:
                @pl.when(kk >= 1)
                def _():
                    drain_scatters(p2, (p + 2) % 4)
            wait_idx(1 - p2, (p + 1) % 4)
            adjust_and_gather(1 - p2, (p + 1) % 4)
            if p == 3:
                @pl.when(kk < CH2 // 4 - 1)
                def _():
                    fire_idx(k + 2, p2, (p + 2) % 4)
            else:
                fire_idx(k + 2, p2, (p + 2) % 4)
            compute(p2, p)
            fire_scatters(p2, p)
        return carry

    lax.fori_loop(0, CH2 // 4, quad, 0)
    wait_gathers(0)
    drain_scatters(0, 2)
    compute(0, 0)
    fire_scatters(0, 0)
    drain_scatters(1, 3)
    drain_scatters(0, 0)
    plsc.subcore_barrier()
    pltpu.sync_copy(acc.at[pl.ds(base_n, NPT)],
                    out_hbm.at[c, pl.ds(base_n, NPT)])


def _final_body(p_ref, invd_ref, o_ref):
    inv = invd_ref[...]
    o_ref[...] = jnp.concatenate([p_ref[0] * inv, p_ref[1] * inv], axis=1)


_final = pl.pallas_call(
    _final_body,
    grid=(N // _RB,),
    in_specs=[
        pl.BlockSpec((NC, _RB, DH), lambda i: (0, i, 0)),
        pl.BlockSpec((_RB, 1), lambda i: (i, 0)),
    ],
    out_specs=pl.BlockSpec((_RB, D), lambda i: (i, 0)),
    out_shape=jax.ShapeDtypeStruct((N, D), jnp.float32),
)


def kernel(x, edge_src, edge_dst, edge_classes, W_src, b_src, W_dst, b_dst,
           edge_emb):
    es2 = edge_src.reshape(ROWS, RW)
    ed2 = edge_dst.reshape(ROWS, RW)
    hist = _hist(es2, ed2)
    ht, invs = _prep(x, W_src, W_dst, b_src.reshape(1, D),
                     b_dst.reshape(1, D), hist)
    inv_ns = invs[0]
    inv_nd = invs[1, :N].reshape(N, 1)
    emb2 = edge_emb.reshape(T, NC, DH).transpose(1, 0, 2).reshape(NC, T * DH)
    parts = _main(ht.reshape(NC * 2 * N, DH), es2, ed2, edge_classes,
                  inv_ns, emb2)
    return _final(parts, inv_nd)


# speedup vs baseline: 1.2800x; 1.0182x over previous
import functools

import jax
import jax.numpy as jnp
from jax import lax
from jax.experimental import pallas as pl
from jax.experimental.pallas import tpu as pltpu
from jax.experimental.pallas import tpu_sc as plsc

N = 10000
E = 320000
D = 128
T = 16
DH = D // 2
NP = 10240
NC = 2
NS = 16
NW = NC * NS
RW = 80
ROWS = E // RW
RPT = ROWS // NS
SCH = 5
CHUNKS = RPT // SCH
CE = SCH * RW
NPT = N // NS

_mesh = plsc.VectorSubcoreMesh(core_axis_name="c", subcore_axis_name="s")
_sc_params = pltpu.CompilerParams(use_tc_tiling_on_sc=False,
                                  needs_layout_passes=False)


@functools.partial(
    pl.kernel,
    out_type=jax.ShapeDtypeStruct((NC, 2, NP), jnp.float32),
    mesh=_mesh,
    scratch_types=[
        pltpu.VMEM_SHARED((NP,), jnp.float32),
        pltpu.VMEM_SHARED((NP,), jnp.float32),
        pltpu.VMEM((SCH, RW), jnp.int32),
        pltpu.VMEM((RW,), jnp.float32),
        pltpu.VMEM((NP // NS,), jnp.float32),
    ],
    compiler_params=_sc_params,
)
def _hist(es_hbm, ed_hbm, out_hbm, hs_sp, hd_sp, idxb, onesb, zb):
    c = lax.axis_index("c")
    s = lax.axis_index("s")
    wid = s * NC + c
    for i in range(RW // 16):
        onesb[pl.ds(16 * i, 16)] = jnp.ones((16,), jnp.float32)
    for i in range(NP // NS // 16):
        zb[pl.ds(16 * i, 16)] = jnp.zeros((16,), jnp.float32)
    zoff = s * (NP // NS)
    pltpu.sync_copy(zb, hs_sp.at[pl.ds(zoff, NP // NS)])
    pltpu.sync_copy(zb, hd_sp.at[pl.ds(zoff, NP // NS)])
    plsc.subcore_barrier()

    def chunk(k, carry):
        rb = wid * (ROWS // NW) + k * SCH
        pltpu.sync_copy(es_hbm.at[pl.ds(rb, SCH)], idxb)
        for i in range(SCH):
            pltpu.sync_copy(onesb, hs_sp.at[idxb.at[i]], add=True)
        pltpu.sync_copy(ed_hbm.at[pl.ds(rb, SCH)], idxb)
        for i in range(SCH):
            pltpu.sync_copy(onesb, hd_sp.at[idxb.at[i]], add=True)
        return carry

    lax.fori_loop(0, ROWS // NW // SCH, chunk, 0)
    plsc.subcore_barrier()
    pltpu.sync_copy(hs_sp.at[pl.ds(zoff, NP // NS)],
                    out_hbm.at[c, 0, pl.ds(zoff, NP // NS)])
    pltpu.sync_copy(hd_sp.at[pl.ds(zoff, NP // NS)],
                    out_hbm.at[c, 1, pl.ds(zoff, NP // NS)])


_RB = 1000


def _prep_body(x_ref, ws_ref, wd_ref, bs_ref, bd_ref, emb_ref, hist_ref,
               hs_out, hde_out, inv_out):
    x = x_ref[...]
    hs = jnp.dot(x, ws_ref[...], preferred_element_type=jnp.float32) \
        + bs_ref[...]
    hd = jnp.dot(x, wd_ref[...], preferred_element_type=jnp.float32) \
        + bd_ref[...]
    hs_out[0] = hs[:, :DH]
    hs_out[1] = hs[:, DH:]
    for cc in range(NC):
        hde_out[cc] = hd[:, cc * DH:(cc + 1) * DH][:, None, :] \
            + emb_ref[cc][None, :, :]

    @pl.when(pl.program_id(0) == 0)
    def _():
        deg = hist_ref[0] + hist_ref[1]
        inv_out[...] = lax.rsqrt(jnp.maximum(deg, 1.0))


_prep = pl.pallas_call(
    _prep_body,
    grid=(N // _RB,),
    in_specs=[
        pl.BlockSpec((_RB, D), lambda i: (i, 0)),
        pl.BlockSpec((D, D), lambda i: (0, 0)),
        pl.BlockSpec((D, D), lambda i: (0, 0)),
        pl.BlockSpec((1, D), lambda i: (0, 0)),
        pl.BlockSpec((1, D), lambda i: (0, 0)),
        pl.BlockSpec((NC, T, DH), lambda i: (0, 0, 0)),
        pl.BlockSpec((NC, 2, NP), lambda i: (0, 0, 0)),
    ],
    out_specs=[
        pl.BlockSpec((NC, _RB, DH), lambda i: (0, i, 0)),
        pl.BlockSpec((NC, _RB, T, DH), lambda i: (0, i, 0, 0)),
        pl.BlockSpec((2, NP), lambda i: (0, 0)),
    ],
    out_shape=[
        jax.ShapeDtypeStruct((NC, N, DH), jnp.float32),
        jax.ShapeDtypeStruct((NC, N, T, DH), jnp.float32),
        jax.ShapeDtypeStruct((2, NP), jnp.float32),
    ],
)


SCH2 = 2
CE2 = SCH2 * RW
CH2 = RPT // SCH2
GPC = CE2 // 16


@functools.partial(
    pl.kernel,
    out_type=jax.ShapeDtypeStruct((NC, N, DH), jnp.float32),
    mesh=_mesh,
    scratch_types=(
        [pltpu.VMEM((SCH2, RW), jnp.int32) for _ in range(2)]
        + [pltpu.VMEM((SCH2, RW), jnp.int32) for _ in range(2)]
        + [pltpu.VMEM((SCH2, RW), jnp.int32) for _ in range(4)]
        + [pltpu.VMEM((SCH2, RW), jnp.int32) for _ in range(2)]
        + [pltpu.VMEM((CE2,), jnp.int32) for _ in range(4)]
        + [pltpu.VMEM((CE2,), jnp.float32) for _ in range(2)]
        + [pltpu.VMEM((CE2, DH), jnp.float32) for _ in range(2)]
        + [pltpu.VMEM((CE2, DH), jnp.float32) for _ in range(2)]
        + [pltpu.VMEM((CE2, DH), jnp.float32) for _ in range(2)]
        + [pltpu.VMEM_SHARED((N, DH), jnp.float32)]
        + [pltpu.SemaphoreType.DMA for _ in range(6)]
    ),
    compiler_params=_sc_params,
)
def _main(hs_hbm, hde_hbm, es_hbm, ed_hbm, ec_hbm, inv_hbm, out_hbm,
          sidx0, sidx1, sidxa0, sidxa1, didx0, didx1, didx2_, didx3,
          didxa0, didxa1, cvec0, cvec1, cvec2, cvec3, invv0, invv1,
          srcb0, srcb1, dstb0, dstb1, msgb0, msgb1, acc,
          semg0, semg1, sems0, sems1, semi0, semi1):
    SIDX = [sidx0, sidx1]
    SIDXA = [sidxa0, sidxa1]
    DIDX = [didx0, didx1, didx2_, didx3]
    DIDXA = [didxa0, didxa1]
    CVEC = [cvec0, cvec1, cvec2, cvec3]
    INVV = [invv0, invv1]
    SRCB = [srcb0, srcb1]
    DSTB = [dstb0, dstb1]
    MSGB = [msgb0, msgb1]
    SEMG = [semg0, semg1]
    SEMS = [sems0, sems1]
    SEMI = [semi0, semi1]

    c = lax.axis_index("c")
    s = lax.axis_index("s")

    def zr(r, carry):
        for j in range(DH // 16):
            msgb0[r, pl.ds(16 * j, 16)] = jnp.zeros((16,), jnp.float32)
        return carry

    lax.fori_loop(0, CE2, zr, 0)
    base_n = s * NPT
    for r0 in range(0, NPT - CE2 + 1, CE2):
        pltpu.sync_copy(msgb0.at[pl.ds(0, CE2)],
                        acc.at[pl.ds(base_n + r0, CE2)])
    rem = NPT % CE2
    if rem:
        pltpu.sync_copy(msgb0.at[pl.ds(0, rem)],
                        acc.at[pl.ds(base_n + NPT - rem, rem)])
    plsc.subcore_barrier()

    soff = c * N
    doff = c * (N * T)

    def fire_idx(k, pi, d4):
        rb = s * RPT + k * SCH2
        eb = rb * RW
        pltpu.async_copy(es_hbm.at[pl.ds(rb, SCH2)], SIDX[pi], SEMI[pi])
        pltpu.async_copy(ed_hbm.at[pl.ds(rb, SCH2)], DIDX[d4], SEMI[pi])
        pltpu.async_copy(ec_hbm.at[pl.ds(eb, CE2)], CVEC[d4], SEMI[pi])

    def wait_idx(pi, d4):
        pltpu.make_async_copy(es_hbm.at[pl.ds(0, SCH2)],
                              SIDX[pi], SEMI[pi]).wait()
        pltpu.make_async_copy(ed_hbm.at[pl.ds(0, SCH2)],
                              DIDX[d4], SEMI[pi]).wait()
        pltpu.make_async_copy(ec_hbm.at[pl.ds(0, CE2)],
                              CVEC[d4], SEMI[pi]).wait()

    def adjust_and_gather(p, d4):
        for i in range(SCH2):
            for q in range(RW // 16):
                SIDXA[p][i, pl.ds(16 * q, 16)] = \
                    SIDX[p][i, pl.ds(16 * q, 16)] + soff
                DIDXA[p][i, pl.ds(16 * q, 16)] = \
                    DIDX[d4][i, pl.ds(16 * q, 16)] * T \
                    + CVEC[d4][pl.ds(RW * i + 16 * q, 16)] + doff
        for i in range(SCH2):
            pltpu.async_copy(hs_hbm.at[SIDXA[p].at[i]],
                             SRCB[p].at[pl.ds(RW * i, RW)], SEMG[p])
            pltpu.async_copy(hde_hbm.at[DIDXA[p].at[i]],
                             DSTB[p].at[pl.ds(RW * i, RW)], SEMG[p])
            pltpu.async_copy(inv_hbm.at[SIDX[p].at[i]],
                             INVV[p].at[pl.ds(RW * i, RW)], SEMG[p])

    def wait_gathers(p):
        for i in range(SCH2):
            pltpu.make_async_copy(hs_hbm.at[SIDXA[p].at[i]],
                                  SRCB[p].at[pl.ds(RW * i, RW)],
                                  SEMG[p]).wait()
            pltpu.make_async_copy(hde_hbm.at[DIDXA[p].at[i]],
                                  DSTB[p].at[pl.ds(RW * i, RW)],
                                  SEMG[p]).wait()
            pltpu.make_async_copy(inv_hbm.at[SIDX[p].at[i]],
                                  INVV[p].at[pl.ds(RW * i, RW)],
                                  SEMG[p]).wait()

    def drain_scatters(p, d4):
        for i in range(SCH2):
            pltpu.make_async_copy(MSGB[p].at[pl.ds(RW * i, RW)],
                                  acc.at[DIDX[d4].at[i]], SEMS[p]).wait()

    def bcast(vec, r):
        idx = jnp.full((16, 1), r, jnp.int32)
        return lax.gather(
            vec, idx,
            lax.GatherDimensionNumbers(offset_dims=(),
                                       collapsed_slice_dims=(0,),
                                       start_index_map=(0,)),
            (1,), mode=lax.GatherScatterMode.PROMISE_IN_BOUNDS)

    def compute(p, c4):
        del c4
        def group(g, gc):
            inv_v = INVV[p][pl.ds(16 * g, 16)]
            for r in range(16):
                inv_b = bcast(inv_v, r)
                b = 16 * g + r
                for j in range(DH // 16):
                    sv = SRCB[p][b, pl.ds(16 * j, 16)]
                    dv = DSTB[p][b, pl.ds(16 * j, 16)]
                    m = jnp.maximum(sv + dv, 0.0) * inv_b
                    MSGB[p][b, pl.ds(16 * j, 16)] = m
            return gc

        lax.fori_loop(0, GPC, group, 0)

    def fire_scatters(p, d4):
        for i in range(SCH2):
            pltpu.async_copy(MSGB[p].at[pl.ds(RW * i, RW)],
                             acc.at[DIDX[d4].at[i]], SEMS[p], add=True)

    fire_idx(0, 0, 0)
    wait_idx(0, 0)
    adjust_and_gather(0, 0)
    fire_idx(1, 1, 1)

    def quad(kk, carry):
        for p in range(4):
            k = kk * 4 + p
            p2 = p % 2
            wait_gathers(p2)
            if p >= 2:
                drain_scatters(p2, (p - 2) % 4)
            else:
                @pl.when(kk >= 1)
                def _():
                    drain_scatters(p2, (p + 2) % 4)
            wait_idx(1 - p2, (p + 1) % 4)
            adjust_and_gather(1 - p2, (p + 1) % 4)
            if p == 3:
                @pl.when(kk < CH2 // 4 - 1)
                def _():
                    fire_idx(k + 2, p2, (p + 2) % 4)
            else:
                fire_idx(k + 2, p2, (p + 2) % 4)
            compute(p2, p)
            fire_scatters(p2, p)
        return carry

    lax.fori_loop(0, CH2 // 4, quad, 0)
    wait_gathers(0)
    drain_scatters(0, 2)
    compute(0, 0)
    fire_scatters(0, 0)
    drain_scatters(1, 3)
    drain_scatters(0, 0)
    plsc.subcore_barrier()
    pltpu.sync_copy(acc.at[pl.ds(base_n, NPT)],
                    out_hbm.at[c, pl.ds(base_n, NPT)])


def _final_body(p_ref, invd_ref, o_ref):
    inv = invd_ref[...]
    o_ref[...] = jnp.concatenate([p_ref[0] * inv, p_ref[1] * inv], axis=1)


_final = pl.pallas_call(
    _final_body,
    grid=(N // _RB,),
    in_specs=[
        pl.BlockSpec((NC, _RB, DH), lambda i: (0, i, 0)),
        pl.BlockSpec((_RB, 1), lambda i: (i, 0)),
    ],
    out_specs=pl.BlockSpec((_RB, D), lambda i: (i, 0)),
    out_shape=jax.ShapeDtypeStruct((N, D), jnp.float32),
)


def kernel(x, edge_src, edge_dst, edge_classes, W_src, b_src, W_dst, b_dst,
           edge_emb):
    es2 = edge_src.reshape(ROWS, RW)
    ed2 = edge_dst.reshape(ROWS, RW)
    hist = _hist(es2, ed2)
    emb3 = edge_emb.reshape(T, NC, DH).transpose(1, 0, 2)
    hs_tab, hde_tab, invs = _prep(x, W_src, W_dst, b_src.reshape(1, D),
                                  b_dst.reshape(1, D), emb3, hist)
    inv_ns = invs[0]
    inv_nd = invs[1, :N].reshape(N, 1)
    parts = _main(hs_tab.reshape(NC * N, DH),
                  hde_tab.reshape(NC * N * T, DH), es2, ed2, edge_classes,
                  inv_ns)
    return _final(parts, inv_nd)
